# baseline (speedup 1.0000x reference)
import functools
import jax
import jax.numpy as jnp
from jax import lax
from jax.experimental import pallas as pl
from jax.experimental.pallas import tpu as pltpu
try:
    from jax.experimental.pallas import tpu_sc as plsc
except ImportError:
    plsc = None

_B, _DIM, _L = 8, 512, 512
_NQ, _G, _CS = 4, 2, 1024
_DPG = _DIM // _G
_F32 = jnp.float32
_BF16 = jnp.bfloat16


def _body(x_r, w3_r, pw_r, po_r, cb_r, cbm2_r, hiT_r, midT_r, loT_r,
          cvb_r, prb_r, pob_r, q_r, idx_r, loss_r):
    xb = x_r[0]
    m0 = jnp.dot(w3_r[0], xb, preferred_element_type=_F32)
    m1 = jnp.dot(w3_r[1], xb, preferred_element_type=_F32)
    m2 = jnp.dot(w3_r[2], xb, preferred_element_type=_F32)
    zcol = jnp.zeros((_DIM, 1), _F32)
    y = (m1
         + jnp.concatenate([zcol, m0[:, :-1]], axis=1)
         + jnp.concatenate([m2[:, 1:], zcol], axis=1)
         + cvb_r[...])
    z = jnp.dot(pw_r[...], y.astype(_BF16), preferred_element_type=_F32) + prb_r[...]

    iota = lax.broadcasted_iota(jnp.int32, (_CS, _L), 0)
    qcat_parts, loss_vals, idx_vals = [], [], []
    for g in range(_G):
        r0 = z[_DPG * g:_DPG * (g + 1), :]
        r = r0
        for qi in range(_NQ):
            cb_f = cb_r[g, qi]
            c2 = jnp.sum(cb_f * cb_f, axis=1, keepdims=True)
            t = jnp.dot(cbm2_r[g, qi], r.astype(_BF16),
                        preferred_element_type=_F32) + c2
            tmin = jnp.min(t, axis=0, keepdims=True)
            idxr = jnp.min(jnp.where(t <= tmin, iota, _CS), axis=0, keepdims=True)
            oh = (iota == idxr).astype(_BF16)
            qv = ((jnp.dot(hiT_r[g, qi], oh, preferred_element_type=_F32)
                   + jnp.dot(midT_r[g, qi], oh, preferred_element_type=_F32)
                   * _F32(2.0 ** -16))
                  + jnp.dot(loT_r[g, qi], oh, preferred_element_type=_F32)
                  * _F32(2.0 ** -32))
            r = r - qv
            loss_vals.append(jnp.sum(r * r))
            idx_vals.append(idxr)
        qcat_parts.append(r0 - r)
    qcat = jnp.concatenate(qcat_parts, axis=0)
    q_r[0] = jnp.dot(po_r[...], qcat.astype(_BF16),
                     preferred_element_type=_F32) + pob_r[...]
    idx_r[0] = jnp.concatenate(idx_vals, axis=0)
    loss_r[0] = jnp.concatenate(
        [jnp.broadcast_to(v, (1, 128)) for v in loss_vals], axis=0)


def kernel(x, conv_w, conv_b, pre_w, pre_b, codebooks, post_w, post_b):
    xb = x.astype(_BF16)
    w3 = jnp.transpose(conv_w, (2, 0, 1)).astype(_BF16)
    pw = pre_w.astype(_BF16)
    po = post_w.astype(_BF16)
    cbm2 = (codebooks.astype(_BF16)) * _BF16(-2.0)
    cbT = jnp.transpose(codebooks, (0, 1, 3, 2))
    hiT = cbT.astype(_BF16)
    remT = cbT - lax.optimization_barrier(hiT).astype(_F32)
    midT = (remT * _F32(2.0 ** 16)).astype(_BF16)
    remT2 = remT - lax.optimization_barrier(midT).astype(_F32) * _F32(2.0 ** -16)
    loT = (remT2 * _F32(2.0 ** 32)).astype(_BF16)
    cvb = conv_b.reshape(_DIM, 1)
    prb = pre_b.reshape(_DIM, 1)
    pob = post_b.reshape(_DIM, 1)

    const = lambda *blk: pl.BlockSpec(blk, lambda b: (0,) * len(blk))
    q, idx_out, loss_out = pl.pallas_call(
        _body,
        grid=(_B,),
        in_specs=[
            pl.BlockSpec((1, _DIM, _L), lambda b: (b, 0, 0)),
            const(3, _DIM, _DIM),
            const(_DIM, _DIM),
            const(_DIM, _DIM),
            const(_G, _NQ, _CS, _DPG),
            const(_G, _NQ, _CS, _DPG),
            const(_G, _NQ, _DPG, _CS),
            const(_G, _NQ, _DPG, _CS),
            const(_G, _NQ, _DPG, _CS),
            const(_DIM, 1),
            const(_DIM, 1),
            const(_DIM, 1),
        ],
        out_specs=[
            pl.BlockSpec((1, _DIM, _L), lambda b: (b, 0, 0)),
            pl.BlockSpec((1, _G * _NQ, _L), lambda b: (b, 0, 0)),
            pl.BlockSpec((1, _G * _NQ, 128), lambda b: (b, 0, 0)),
        ],
        out_shape=[
            jax.ShapeDtypeStruct((_B, _DIM, _L), _F32),
            jax.ShapeDtypeStruct((_B, _G * _NQ, _L), jnp.int32),
            jax.ShapeDtypeStruct((_B, _G * _NQ, 128), _F32),
        ],
    )(xb, w3, pw, po, codebooks, cbm2, hiT, midT, loT, cvb, prb, pob)

    indices = (jnp.transpose(idx_out, (1, 0, 2))
               .reshape(_G, _NQ, _B, _L)
               .transpose(0, 2, 3, 1))
    losses = (loss_out[:, :, 0].sum(axis=0)
              .reshape(_G, _NQ) / (_B * _L * _DPG))
    tbl = codebooks.reshape(_G * _NQ * _CS, _DPG)
    offs = (jnp.arange(_G * _NQ, dtype=jnp.int32) * _CS)[None, :, None]
    idx_flat = (idx_out + offs).transpose(0, 1, 2).reshape(-1)
    gathered = _sc_gather(tbl, idx_flat)
    return q, indices, losses, gathered


def _sc_gather(tbl, idx):
    n = idx.shape[0]
    mesh = plsc.VectorSubcoreMesh(core_axis_name="c", subcore_axis_name="s")
    NW, CH = 32, 256
    per_w = n // NW
    nch = per_w // CH

    @functools.partial(
        pl.kernel, mesh=mesh,
        out_type=jax.ShapeDtypeStruct((n, _DPG), _F32),
        scratch_types=[
            pltpu.VMEM((CH,), jnp.int32),
            pltpu.VMEM((CH, _DPG), _F32),
            pltpu.SemaphoreType.DMA,
        ],
    )
    def k(tbl_hbm, idx_hbm, out_hbm, idx_v, rows_v, sem):
        wid = lax.axis_index("s") * 2 + lax.axis_index("c")
        base = wid * per_w
        for c in range(nch):
            off = base + c * CH
            pltpu.sync_copy(idx_hbm.at[pl.ds(off, CH)], idx_v)
            pltpu.async_copy(tbl_hbm.at[idx_v], rows_v, sem).wait()
            pltpu.sync_copy(rows_v, out_hbm.at[pl.ds(off, CH)])

    return k(tbl, idx)
